# baseline (speedup 1.0000x reference)
import functools

import jax
import jax.numpy as jnp
from jax import lax
from jax.experimental import pallas as pl
from jax.experimental.pallas import tpu as pltpu, tpu_sc as plsc

N_POINTS = 4096
HIDDEN = 1024
Z_MIN, Z_MAX = -3.0, 3.0
C = 1.0
DT = (Z_MAX - Z_MIN) / (N_POINTS - 1)
INV_DT = 1.0 / DT

COLS = 512
GRID = N_POINTS // COLS

NB = 16384
NW = 32
CHUNK = NB // NW
LANES = 16

_PREC = lax.Precision.HIGHEST
_MLP_PREC = lax.Precision.DEFAULT


def _mlp_body(W1_ref, b1_ref, W2_ref, b2_ref, W3_ref, b3_ref,
              w_ref, F_ref, wacc_ref):
    i = pl.program_id(0)
    col = (lax.broadcasted_iota(jnp.int32, (1, COLS), 1) + i * COLS)
    t = Z_MIN + col.astype(jnp.float32) * DT
    h = jnp.tanh(W1_ref[...] * t + b1_ref[...])
    h = jnp.tanh(
        lax.dot_general(W2_ref[...], h, (((1,), (0,)), ((), ())),
                        preferred_element_type=jnp.float32,
                        precision=_MLP_PREC)
        + b2_ref[...])
    g8 = lax.dot_general(W3_ref[...], h, (((1,), (0,)), ((), ())),
                         preferred_element_type=jnp.float32,
                         precision=_MLP_PREC)
    g = g8[0:1, :] + b3_ref[0, 0]
    w = jnp.exp(jnp.clip(g, -C, C))
    w_ref[...] = w.reshape(1, 1, COLS)
    wacc_ref[pl.ds(i, 1), :] = w

    @pl.when(i == GRID - 1)
    def _scan():
        X = wacc_ref[...]
        k = lax.broadcasted_iota(jnp.int32, (COLS, COLS), 0)
        j = lax.broadcasted_iota(jnp.int32, (COLS, COLS), 1)
        U = (k < j).astype(jnp.float32)
        row_excl = lax.dot_general(X, U, (((1,), (0,)), ((), ())),
                                   preferred_element_type=jnp.float32,
                                   precision=_PREC)
        r = jnp.broadcast_to(jnp.sum(X, axis=1, keepdims=True), (GRID, 8))
        ii = lax.broadcasted_iota(jnp.int32, (GRID, GRID), 0)
        mm = lax.broadcasted_iota(jnp.int32, (GRID, GRID), 1)
        V = (mm < ii).astype(jnp.float32)
        roffs = lax.dot_general(V, r[:, :GRID], (((1,), (0,)), ((), ())),
                                preferred_element_type=jnp.float32,
                                precision=_PREC)
        E = row_excl + roffs[:, 0:1]
        w00 = X[0, 0]
        F_ref[...] = DT * E + (0.5 * DT) * (X - w00)


_sc_mesh = plsc.VectorSubcoreMesh(core_axis_name="c", subcore_axis_name="s")


@functools.partial(
    pl.kernel,
    mesh=_sc_mesh,
    compiler_params=pltpu.CompilerParams(needs_layout_passes=False),
    out_type=jax.ShapeDtypeStruct((NB,), jnp.float32),
    scratch_types=[
        pltpu.VMEM((N_POINTS,), jnp.float32),
        pltpu.VMEM((N_POINTS,), jnp.float32),
        pltpu.VMEM((CHUNK,), jnp.float32),
        pltpu.VMEM((CHUNK,), jnp.float32),
    ],
)
def _gather_lerp(F_hbm, w_hbm, z_hbm, out_hbm, F_v, w_v, z_v, o_v):
    wid = lax.axis_index("s") * 2 + lax.axis_index("c")
    base = wid * CHUNK
    pltpu.sync_copy(F_hbm, F_v)
    pltpu.sync_copy(w_hbm, w_v)
    pltpu.sync_copy(z_hbm.at[pl.ds(base, CHUNK)], z_v)
    w_head = w_v[pl.ds(0, LANES)]
    w_tail = w_v[pl.ds(N_POINTS - LANES, LANES)]
    F_tail = F_v[pl.ds(N_POINTS - LANES, LANES)]
    w0 = w_head[0]
    wN = w_tail[LANES - 1]
    FN = F_tail[LANES - 1]

    def body(i, carry):
        zv = z_v[pl.ds(i * LANES, LANES)]
        pos = (zv - Z_MIN) * INV_DT
        idx = jnp.clip(pos.astype(jnp.int32), 0, N_POINTS - 2)
        frac = jnp.clip(pos - idx.astype(jnp.float32), 0.0, 1.0)
        F_lo = plsc.load_gather(F_v, [idx])
        w_lo = plsc.load_gather(w_v, [idx])
        F_mid = F_lo + frac * w_lo * DT
        F_low = (zv - Z_MIN) * w0
        F_high = FN + (zv - Z_MAX) * wN
        fz = jnp.where(zv < Z_MIN, F_low, jnp.where(zv > Z_MAX, F_high, F_mid))
        o_v[pl.ds(i * LANES, LANES)] = fz
        return carry

    lax.fori_loop(0, CHUNK // LANES, body, 0)
    pltpu.sync_copy(o_v, out_hbm.at[pl.ds(base, CHUNK)])


def _norm_body(Fz_ref, a_ref, b_ref, out_ref):
    X = Fz_ref[...]
    mu = jnp.sum(X) * (1.0 / NB)
    d = X - mu
    var = jnp.sum(d * d) * (1.0 / (NB - 1))
    sigma = jnp.maximum(jnp.sqrt(var), 0.001)
    ar = a_ref[0, 0]
    a = jnp.maximum(ar, 0.0) + jnp.log(1.0 + jnp.exp(-jnp.abs(ar))) + 0.001
    out_ref[...] = (a / (sigma + 1e-6)) * d + b_ref[0, 0]


def kernel(z, W1, b1, W2, b2, W3, b3, a_raw, b_out):
    H = HIDDEN
    b1c = b1.reshape(H, 1)
    b2c = b2.reshape(H, 1)
    W3r = jnp.broadcast_to(W3.reshape(1, H), (8, H))
    b3r = b3.reshape(1, 1)

    full = lambda shape: pl.BlockSpec(shape, lambda i: (0, 0))
    w_row, F_row = pl.pallas_call(
        _mlp_body,
        grid=(GRID,),
        in_specs=[full((H, 1)), full((H, 1)), full((H, H)), full((H, 1)),
                  full((8, H)), full((1, 1))],
        out_specs=[pl.BlockSpec((1, 1, COLS), lambda i: (i, 0, 0)),
                   full((GRID, COLS))],
        out_shape=[jax.ShapeDtypeStruct((GRID, 1, COLS), jnp.float32),
                   jax.ShapeDtypeStruct((GRID, COLS), jnp.float32)],
        scratch_shapes=[pltpu.VMEM((GRID, COLS), jnp.float32)],
    )(W1, b1c, W2, b2c, W3r, b3r)

    Fz = _gather_lerp(F_row.reshape(N_POINTS), w_row.reshape(N_POINTS),
                      z.reshape(NB))

    out2d = pl.pallas_call(
        _norm_body,
        out_shape=jax.ShapeDtypeStruct((128, 128), jnp.float32),
    )(Fz.reshape(128, 128), a_raw.reshape(1, 1), b_out.reshape(1, 1))
    return out2d.reshape(z.shape)


# speedup vs baseline: 1.1137x; 1.1137x over previous
import functools

import jax
import jax.numpy as jnp
from jax import lax
from jax.experimental import pallas as pl
from jax.experimental.pallas import tpu as pltpu, tpu_sc as plsc

N_POINTS = 4096
HIDDEN = 1024
Z_MIN, Z_MAX = -3.0, 3.0
C = 1.0
DT = (Z_MAX - Z_MIN) / (N_POINTS - 1)
INV_DT = 1.0 / DT

COLS = 512
GRID = N_POINTS // COLS

NB = 16384
NW = 32
CHUNK = NB // NW
LANES = 16

_PREC = lax.Precision.HIGHEST
_MLP_PREC = lax.Precision.DEFAULT


def _mlp_body(W1_ref, b1_ref, W2_ref, b2_ref, W3_ref, b3_ref,
              w_ref, F_ref, wacc_ref):
    i = pl.program_id(0)
    row = lax.broadcasted_iota(jnp.int32, (COLS, 1), 0) + i * COLS
    t = Z_MIN + row.astype(jnp.float32) * DT
    h = jnp.tanh(
        lax.dot_general(t, W1_ref[...], (((1,), (1,)), ((), ())),
                        preferred_element_type=jnp.float32,
                        precision=_MLP_PREC)
        + b1_ref[...])
    h = jnp.tanh(
        lax.dot_general(h, W2_ref[...], (((1,), (1,)), ((), ())),
                        preferred_element_type=jnp.float32,
                        precision=_MLP_PREC)
        + b2_ref[...])
    W3b = jnp.broadcast_to(W3_ref[...], (8, HIDDEN))
    g8 = lax.dot_general(W3b, h, (((1,), (1,)), ((), ())),
                         preferred_element_type=jnp.float32,
                         precision=_MLP_PREC)
    g = g8[0:1, :] + b3_ref[0, 0]
    w = jnp.exp(jnp.clip(g, -C, C))
    wacc_ref[pl.ds(i * (COLS // 128), COLS // 128), :] = w.reshape(COLS // 128, 128)

    @pl.when(i == GRID - 1)
    def _scan():
        X = wacc_ref[...]
        w_ref[...] = X
        k = lax.broadcasted_iota(jnp.int32, (128, 128), 0)
        j = lax.broadcasted_iota(jnp.int32, (128, 128), 1)
        U = (k < j).astype(jnp.float32)
        row_excl = lax.dot_general(X, U, (((1,), (0,)), ((), ())),
                                   preferred_element_type=jnp.float32,
                                   precision=_PREC)
        r = jnp.broadcast_to(jnp.sum(X, axis=1, keepdims=True), (32, 8))
        ii = lax.broadcasted_iota(jnp.int32, (32, 32), 0)
        mm = lax.broadcasted_iota(jnp.int32, (32, 32), 1)
        V = (mm < ii).astype(jnp.float32)
        roffs = lax.dot_general(V, r[:, :8], (((1,), (0,)), ((), ())),
                                preferred_element_type=jnp.float32,
                                precision=_PREC)
        E = row_excl + roffs[:, 0:1]
        w00 = X[0, 0]
        F_ref[...] = DT * E + (0.5 * DT) * (X - w00)


_sc_mesh = plsc.VectorSubcoreMesh(core_axis_name="c", subcore_axis_name="s")


@functools.partial(
    pl.kernel,
    mesh=_sc_mesh,
    compiler_params=pltpu.CompilerParams(needs_layout_passes=False),
    out_type=jax.ShapeDtypeStruct((128, 128), jnp.float32),
    scratch_types=[
        pltpu.VMEM((N_POINTS,), jnp.float32),
        pltpu.VMEM((N_POINTS,), jnp.float32),
        pltpu.VMEM((CHUNK,), jnp.float32),
        pltpu.VMEM((CHUNK // 128, 128), jnp.float32),
    ],
)
def _gather_lerp(F_hbm, w_hbm, z_hbm, out_hbm, F_v, w_v, z_v, o_v):
    wid = lax.axis_index("s") * 2 + lax.axis_index("c")
    base = wid * CHUNK
    pltpu.sync_copy(F_hbm, F_v)
    pltpu.sync_copy(w_hbm, w_v)
    pltpu.sync_copy(z_hbm.at[pl.ds(base, CHUNK)], z_v)
    w_head = w_v[pl.ds(0, LANES)]
    w_tail = w_v[pl.ds(N_POINTS - LANES, LANES)]
    F_tail = F_v[pl.ds(N_POINTS - LANES, LANES)]
    w0 = w_head[0]
    wN = w_tail[LANES - 1]
    FN = F_tail[LANES - 1]

    def body(i, carry):
        zv = z_v[pl.ds(i * LANES, LANES)]
        pos = (zv - Z_MIN) * INV_DT
        idx = jnp.clip(pos.astype(jnp.int32), 0, N_POINTS - 2)
        frac = jnp.clip(pos - idx.astype(jnp.float32), 0.0, 1.0)
        F_lo = plsc.load_gather(F_v, [idx])
        w_lo = plsc.load_gather(w_v, [idx])
        F_mid = F_lo + frac * w_lo * DT
        F_low = (zv - Z_MIN) * w0
        F_high = FN + (zv - Z_MAX) * wN
        fz = jnp.where(zv < Z_MIN, F_low, jnp.where(zv > Z_MAX, F_high, F_mid))
        o_v[i // 8, pl.ds((i % 8) * LANES, LANES)] = fz
        return carry

    lax.fori_loop(0, CHUNK // LANES, body, 0)
    pltpu.sync_copy(o_v, out_hbm.at[pl.ds(wid * (CHUNK // 128), CHUNK // 128), :])


def _norm_body(Fz_ref, a_ref, b_ref, out_ref):
    X = Fz_ref[...]
    mu = jnp.sum(X) * (1.0 / NB)
    d = X - mu
    var = jnp.sum(d * d) * (1.0 / (NB - 1))
    sigma = jnp.maximum(jnp.sqrt(var), 0.001)
    ar = a_ref[0, 0]
    a = jnp.maximum(ar, 0.0) + jnp.log(1.0 + jnp.exp(-jnp.abs(ar))) + 0.001
    out_ref[...] = (a / (sigma + 1e-6)) * d + b_ref[0, 0]


def kernel(z, W1, b1, W2, b2, W3, b3, a_raw, b_out):
    H = HIDDEN
    b1r = b1.reshape(1, H)
    b2r = b2.reshape(1, H)
    b3r = b3.reshape(1, 1)

    full = lambda shape: pl.BlockSpec(shape, lambda i: (0, 0))
    w2d, F2d = pl.pallas_call(
        _mlp_body,
        grid=(GRID,),
        in_specs=[full((H, 1)), full((1, H)), full((H, H)), full((1, H)),
                  full((1, H)), full((1, 1))],
        out_specs=[full((32, 128)),
                   full((32, 128))],
        out_shape=[jax.ShapeDtypeStruct((32, 128), jnp.float32),
                   jax.ShapeDtypeStruct((32, 128), jnp.float32)],
        scratch_shapes=[pltpu.VMEM((32, 128), jnp.float32)],
    )(W1, b1r, W2, b2r, W3, b3r)

    Fz2d = _gather_lerp(F2d.reshape(N_POINTS), w2d.reshape(N_POINTS),
                        z.reshape(NB))

    out2d = pl.pallas_call(
        _norm_body,
        out_shape=jax.ShapeDtypeStruct((128, 128), jnp.float32),
    )(Fz2d, a_raw.reshape(1, 1), b_out.reshape(1, 1))
    return out2d.reshape(z.shape)


# speedup vs baseline: 1.1501x; 1.0327x over previous
import functools

import jax
import jax.numpy as jnp
from jax import lax
from jax.experimental import pallas as pl
from jax.experimental.pallas import tpu as pltpu, tpu_sc as plsc

N_POINTS = 4096
HIDDEN = 1024
Z_MIN, Z_MAX = -3.0, 3.0
C = 1.0
DT = (Z_MAX - Z_MIN) / (N_POINTS - 1)
INV_DT = 1.0 / DT

COLS = 1024
GRID = N_POINTS // COLS

NB = 16384
NW = 32
CHUNK = NB // NW
LANES = 16

_PREC = lax.Precision.HIGHEST
_MLP_PREC = lax.Precision.DEFAULT


def _mlp_body(W1_ref, b1_ref, W2_ref, b2_ref, W3_ref, b3_ref,
              w_ref, F_ref, wacc_ref):
    i = pl.program_id(0)
    row = lax.broadcasted_iota(jnp.int32, (COLS, 1), 0) + i * COLS
    t = Z_MIN + row.astype(jnp.float32) * DT
    h = jnp.tanh(
        lax.dot_general(t, W1_ref[...], (((1,), (1,)), ((), ())),
                        preferred_element_type=jnp.float32,
                        precision=_MLP_PREC)
        + b1_ref[...])
    h = jnp.tanh(
        lax.dot_general(h.astype(jnp.bfloat16),
                        W2_ref[...].astype(jnp.bfloat16),
                        (((1,), (1,)), ((), ())),
                        preferred_element_type=jnp.float32,
                        precision=_MLP_PREC)
        + b2_ref[...])
    W3b = jnp.broadcast_to(W3_ref[...], (8, HIDDEN))
    g8 = lax.dot_general(W3b, h, (((1,), (1,)), ((), ())),
                         preferred_element_type=jnp.float32,
                         precision=_MLP_PREC)
    g = g8[0:1, :] + b3_ref[0, 0]
    w = jnp.exp(jnp.clip(g, -C, C))
    wacc_ref[pl.ds(i * (COLS // 128), COLS // 128), :] = w.reshape(COLS // 128, 128)

    @pl.when(i == GRID - 1)
    def _scan():
        X = wacc_ref[...]
        w_ref[...] = X
        k = lax.broadcasted_iota(jnp.int32, (128, 128), 0)
        j = lax.broadcasted_iota(jnp.int32, (128, 128), 1)
        U = (k < j).astype(jnp.float32)
        row_excl = lax.dot_general(X, U, (((1,), (0,)), ((), ())),
                                   preferred_element_type=jnp.float32,
                                   precision=_PREC)
        r = jnp.broadcast_to(jnp.sum(X, axis=1, keepdims=True), (32, 8))
        ii = lax.broadcasted_iota(jnp.int32, (32, 32), 0)
        mm = lax.broadcasted_iota(jnp.int32, (32, 32), 1)
        V = (mm < ii).astype(jnp.float32)
        roffs = lax.dot_general(V, r[:, :8], (((1,), (0,)), ((), ())),
                                preferred_element_type=jnp.float32,
                                precision=_PREC)
        E = row_excl + roffs[:, 0:1]
        w00 = X[0, 0]
        F_ref[...] = DT * E + (0.5 * DT) * (X - w00)


_sc_mesh = plsc.VectorSubcoreMesh(core_axis_name="c", subcore_axis_name="s")


@functools.partial(
    pl.kernel,
    mesh=_sc_mesh,
    compiler_params=pltpu.CompilerParams(needs_layout_passes=False),
    out_type=jax.ShapeDtypeStruct((128, 128), jnp.float32),
    scratch_types=[
        pltpu.VMEM((N_POINTS,), jnp.float32),
        pltpu.VMEM((N_POINTS,), jnp.float32),
        pltpu.VMEM((CHUNK // 128, 128), jnp.float32),
        pltpu.VMEM((CHUNK // 128, 128), jnp.float32),
    ],
)
def _gather_lerp(F_hbm, w_hbm, z_hbm, out_hbm, F_v, w_v, z_v, o_v):
    wid = lax.axis_index("s") * 2 + lax.axis_index("c")
    base = wid * CHUNK
    pltpu.sync_copy(F_hbm, F_v)
    pltpu.sync_copy(w_hbm, w_v)
    pltpu.sync_copy(z_hbm.at[pl.ds(wid * (CHUNK // 128), CHUNK // 128), :], z_v)
    w_head = w_v[pl.ds(0, LANES)]
    w_tail = w_v[pl.ds(N_POINTS - LANES, LANES)]
    F_tail = F_v[pl.ds(N_POINTS - LANES, LANES)]
    w0 = w_head[0]
    wN = w_tail[LANES - 1]
    FN = F_tail[LANES - 1]

    def body(i, carry):
        zv = z_v[i // 8, pl.ds((i % 8) * LANES, LANES)]
        pos = (zv - Z_MIN) * INV_DT
        idx = jnp.clip(pos.astype(jnp.int32), 0, N_POINTS - 2)
        frac = jnp.clip(pos - idx.astype(jnp.float32), 0.0, 1.0)
        F_lo = plsc.load_gather(F_v, [idx])
        w_lo = plsc.load_gather(w_v, [idx])
        F_mid = F_lo + frac * w_lo * DT
        F_low = (zv - Z_MIN) * w0
        F_high = FN + (zv - Z_MAX) * wN
        fz = jnp.where(zv < Z_MIN, F_low, jnp.where(zv > Z_MAX, F_high, F_mid))
        o_v[i // 8, pl.ds((i % 8) * LANES, LANES)] = fz
        return carry

    lax.fori_loop(0, CHUNK // LANES, body, 0)
    pltpu.sync_copy(o_v, out_hbm.at[pl.ds(wid * (CHUNK // 128), CHUNK // 128), :])


def _norm_body(Fz_ref, a_ref, b_ref, out_ref):
    X = Fz_ref[...]
    mu = jnp.sum(X) * (1.0 / NB)
    d = X - mu
    var = jnp.sum(d * d) * (1.0 / (NB - 1))
    sigma = jnp.maximum(jnp.sqrt(var), 0.001)
    ar = a_ref[0, 0]
    a = jnp.maximum(ar, 0.0) + jnp.log(1.0 + jnp.exp(-jnp.abs(ar))) + 0.001
    out_ref[...] = (a / (sigma + 1e-6)) * d + b_ref[0, 0]


def kernel(z, W1, b1, W2, b2, W3, b3, a_raw, b_out):
    H = HIDDEN
    b1r = b1.reshape(1, H)
    b2r = b2.reshape(1, H)
    b3r = b3.reshape(1, 1)

    full = lambda shape: pl.BlockSpec(shape, lambda i: (0, 0))
    w2d, F2d = pl.pallas_call(
        _mlp_body,
        grid=(GRID,),
        in_specs=[full((H, 1)), full((1, H)), full((H, H)), full((1, H)),
                  full((1, H)), full((1, 1))],
        out_specs=[full((32, 128)),
                   full((32, 128))],
        out_shape=[jax.ShapeDtypeStruct((32, 128), jnp.float32),
                   jax.ShapeDtypeStruct((32, 128), jnp.float32)],
        scratch_shapes=[pltpu.VMEM((32, 128), jnp.float32)],
    )(W1, b1r, W2, b2r, W3, b3r)

    Fz2d = _gather_lerp(F2d.reshape(N_POINTS), w2d.reshape(N_POINTS),
                        z.reshape(128, 128))

    out2d = pl.pallas_call(
        _norm_body,
        out_shape=jax.ShapeDtypeStruct((128, 128), jnp.float32),
    )(Fz2d, a_raw.reshape(1, 1), b_out.reshape(1, 1))
    return out2d.reshape(z.shape)
